# Initial kernel scaffold; baseline (speedup 1.0000x reference)
import functools

import jax
import jax.numpy as jnp
from jax import lax
from jax.experimental import pallas as pl
from jax.experimental.pallas import tpu as pltpu
from jax.experimental.pallas import tpu_sc as plsc

_NUM_TABLES = 8
_NUM_EMB = 100000
_SHARD = 16
_PRIMES = (31, 43, 59, 61, 73, 97, 103, 113)
_BATCH = 4096
_SEQ = 200

_N_IDS = _BATCH * _SEQ
_NC, _NS, _L = 2, 16, 16
_NW = _NC * _NS
_PER_W = _N_IDS // _NW
_C = 256
_ROWS = _C * _NUM_TABLES
_K = _ROWS // 128
_CHUNKS = _PER_W // _C

_mesh = plsc.VectorSubcoreMesh(
    core_axis_name="c", subcore_axis_name="s", num_cores=_NC, num_subcores=_NS
)


@functools.partial(
    pl.kernel,
    out_type=jax.ShapeDtypeStruct((_N_IDS * _NUM_TABLES, _SHARD), jnp.float32),
    mesh=_mesh,
    scratch_types=[
        pltpu.VMEM((2, _L), jnp.int32),
        pltpu.VMEM((2, _C), jnp.int32),
        pltpu.VMEM((2, _K, 128), jnp.int32),
        pltpu.VMEM((2, _ROWS, _SHARD), jnp.float32),
        pltpu.SemaphoreType.DMA,
        pltpu.SemaphoreType.DMA,
        pltpu.SemaphoreType.DMA,
        pltpu.SemaphoreType.DMA,
    ],
)
def _hash_embed_sc(ids_hbm, pv_hbm, table_hbm, out_hbm,
                   pv_v, ids_v, idx_v, rows_v, sem_i, sem_g0, sem_g1, sem_o):
    wid = lax.axis_index("s") * _NC + lax.axis_index("c")
    base = wid * _PER_W

    pltpu.sync_copy(pv_hbm, pv_v)
    pvec = pv_v[0, :]
    ovec = pv_v[1, :]
    hi = lax.iota(jnp.int32, _L) >> 3

    gsems = (sem_g0, sem_g1)

    def stage_ids(g, buf):
        pltpu.sync_copy(ids_hbm.at[pl.ds(base + g * _C, _C)], ids_v.at[buf])

    def compute_idx(buf):
        def vreg_body(t, _):
            sel = 2 * t + hi
            ids16 = plsc.load_gather(ids_v.at[buf], [sel])
            h = lax.rem((ids16 + 1) * pvec, jnp.int32(_NUM_EMB)) + ovec
            idx_v[buf, t >> 3, pl.ds((t & 7) * _L, _L)] = h
            return 0
        lax.fori_loop(0, _C // 2, vreg_body, 0)

    def fire_gathers(buf, sem):
        return [
            pltpu.async_copy(
                table_hbm.at[idx_v.at[buf, j]],
                rows_v.at[buf, pl.ds(j * 128, 128)],
                sem,
            )
            for j in range(_K)
        ]

    def store_rows(g, buf):
        return pltpu.async_copy(
            rows_v.at[buf],
            out_hbm.at[pl.ds((base + g * _C) * _NUM_TABLES, _ROWS)],
            sem_o,
        )

    stage_ids(0, 0)
    compute_idx(0)
    inflight = fire_gathers(0, gsems[0])

    def chunk_body(g, _):
        return 0

    prev_out = None
    for g in range(1, _CHUNKS):
        buf = g & 1
        pbuf = 1 - buf
        stage_ids(g, buf)
        compute_idx(buf)
        for c in inflight:
            c.wait()
        if prev_out is not None:
            prev_out.wait()
        prev_out = store_rows(g - 1, pbuf)
        inflight = fire_gathers(buf, gsems[buf])

    for c in inflight:
        c.wait()
    if prev_out is not None:
        prev_out.wait()
    store_rows(_CHUNKS - 1, (_CHUNKS - 1) & 1).wait()


def kernel(input_ids, tables):
    ids = input_ids.reshape(-1)
    table = tables.reshape(_NUM_TABLES * _NUM_EMB, _SHARD)
    pv = jnp.array(
        [list(_PRIMES) * 2,
         [i * _NUM_EMB for i in range(_NUM_TABLES)] * 2],
        dtype=jnp.int32,
    )
    out = _hash_embed_sc(ids, pv, table)
    return out.reshape(_BATCH, _SEQ, _NUM_TABLES * _SHARD)


# speedup vs baseline: 36.2089x; 36.2089x over previous
import functools

import jax
import jax.numpy as jnp
from jax import lax
from jax.experimental import pallas as pl
from jax.experimental.pallas import tpu as pltpu
from jax.experimental.pallas import tpu_sc as plsc

_NUM_TABLES = 8
_NUM_EMB = 100000
_SHARD = 16
_PRIMES = (31, 43, 59, 61, 73, 97, 103, 113)
_BATCH = 4096
_SEQ = 200

_N_IDS = _BATCH * _SEQ
_NC, _NS, _L = 2, 16, 16
_NW = _NC * _NS
_PER_W = _N_IDS // _NW
_C = 256
_ROWS = _C * _NUM_TABLES
_IDX_R = _ROWS // 128
_RPT = _C // 128
_CHUNKS = _PER_W // _C

_mesh = plsc.VectorSubcoreMesh(
    core_axis_name="c", subcore_axis_name="s", num_cores=_NC, num_subcores=_NS
)


@functools.partial(
    pl.kernel,
    out_type=jax.ShapeDtypeStruct((_N_IDS, _NUM_TABLES, _SHARD), jnp.float32),
    mesh=_mesh,
    scratch_types=[
        pltpu.VMEM((_C,), jnp.int32),
        pltpu.VMEM((_C,), jnp.int32),
        pltpu.VMEM((_IDX_R, 128), jnp.int32),
        pltpu.VMEM((_IDX_R, 128), jnp.int32),
        pltpu.VMEM((_NUM_TABLES, _C, _SHARD), jnp.float32),
        pltpu.VMEM((_NUM_TABLES, _C, _SHARD), jnp.float32),
        pltpu.SemaphoreType.DMA,
        pltpu.SemaphoreType.DMA,
        pltpu.SemaphoreType.DMA,
        pltpu.SemaphoreType.DMA,
    ],
    compiler_params=pltpu.CompilerParams(use_tc_tiling_on_sc=False),
)
def _hash_embed_sc(ids_hbm, table_hbm, out_hbm,
                   ids_v0, ids_v1, idx_v0, idx_v1, rows_v0, rows_v1,
                   sem_g0, sem_g1, sem_o0, sem_o1):
    wid = lax.axis_index("s") * _NC + lax.axis_index("c")
    base = wid * _PER_W

    rcp = jnp.float32(1.0 / _NUM_EMB)
    ids_b = (ids_v0, ids_v1)
    idx_b = (idx_v0, idx_v1)
    rows_b = (rows_v0, rows_v1)
    gsems = (sem_g0, sem_g1)
    osems = (sem_o0, sem_o1)

    def stage_ids(c, b):
        pltpu.sync_copy(ids_hbm.at[pl.ds(base + c * _C, _C)], ids_b[b])

    def compute_idx(b):
        ids_v, idx_v = ids_b[b], idx_b[b]

        def body(t, _):
            v = ids_v[pl.ds(t * _L, _L)] + 1
            row = t >> 3
            col = (t & 7) * _L
            for i in range(_NUM_TABLES):
                x = v * _PRIMES[i]
                q = (x.astype(jnp.float32) * rcp).astype(jnp.int32)
                r = x - q * _NUM_EMB
                r = jnp.where(r < 0, r + _NUM_EMB, r)
                r = jnp.where(r >= _NUM_EMB, r - _NUM_EMB, r)
                idx_v[_RPT * i + row, pl.ds(col, _L)] = r + i * _NUM_EMB
            return 0
        lax.fori_loop(0, _C // _L, body, 0)

    def gather_descs(b):
        return [
            pltpu.make_async_copy(
                table_hbm.at[idx_b[b].at[_RPT * i + u]],
                rows_b[b].at[i, pl.ds(u * 128, 128)],
                gsems[b],
            )
            for i in range(_NUM_TABLES)
            for u in range(_RPT)
        ]

    def fire_gathers(b):
        for d in gather_descs(b):
            d.start()

    def wait_gathers(b):
        for d in gather_descs(b):
            d.wait()

    def out_descs(c, b):
        return [
            pltpu.make_async_copy(
                rows_b[b].at[i],
                out_hbm.at[pl.ds(base + c * _C, _C), i],
                osems[b],
            )
            for i in range(_NUM_TABLES)
        ]

    def fire_out(c, b):
        for d in out_descs(c, b):
            d.start()

    def wait_out(c, b):
        for d in out_descs(c, b):
            d.wait()

    stage_ids(0, 0)
    compute_idx(0)
    fire_gathers(0)
    stage_ids(1, 1)
    compute_idx(1)
    fire_gathers(1)
    wait_gathers(0)
    fire_out(0, 0)

    def steady(k, _):
        c0 = 2 * k
        stage_ids(c0, 0)
        compute_idx(0)
        wait_out(c0 - 2, 0)
        fire_gathers(0)
        wait_gathers(1)
        fire_out(c0 - 1, 1)
        stage_ids(c0 + 1, 1)
        compute_idx(1)
        wait_out(c0 - 1, 1)
        fire_gathers(1)
        wait_gathers(0)
        fire_out(c0, 0)
        return 0

    lax.fori_loop(1, _CHUNKS // 2, steady, 0)

    wait_gathers(1)
    fire_out(_CHUNKS - 1, 1)
    wait_out(_CHUNKS - 2, 0)
    wait_out(_CHUNKS - 1, 1)


def kernel(input_ids, tables):
    ids = input_ids.reshape(-1)
    table = tables.reshape(_NUM_TABLES * _NUM_EMB, _SHARD)
    out = _hash_embed_sc(ids, table)
    return out.reshape(_BATCH, _SEQ, _NUM_TABLES * _SHARD)
